# baseline (speedup 1.0000x reference)
import functools

import jax
import jax.numpy as jnp
from jax import lax
from jax.experimental import pallas as pl
from jax.experimental.pallas import tpu as pltpu
from jax.experimental.pallas import tpu_sc as plsc

EMB = 64
NC = 2
NS = 16
NW = NC * NS
CHUNK = 128


def _sc_gather_sum(xw_f, xp_f, xs_f, Ww, Wp, Ws):
    total = xw_f.shape[0]
    per_w = total // NW
    n_chunks = per_w // CHUNK
    assert per_w % CHUNK == 0
    mesh = plsc.VectorSubcoreMesh(
        core_axis_name="c", subcore_axis_name="s", num_cores=NC, num_subcores=NS
    )

    @functools.partial(
        pl.kernel,
        out_type=jax.ShapeDtypeStruct((total, EMB), jnp.float32),
        mesh=mesh,
        compiler_params=pltpu.CompilerParams(use_tc_tiling_on_sc=False),
        scratch_types=[
            pltpu.VMEM((CHUNK,), jnp.int32),
            pltpu.VMEM((CHUNK,), jnp.int32),
            pltpu.VMEM((CHUNK,), jnp.int32),
            pltpu.VMEM((CHUNK, EMB), jnp.float32),
            pltpu.VMEM((CHUNK, EMB), jnp.float32),
            pltpu.VMEM((CHUNK, EMB), jnp.float32),
            pltpu.SemaphoreType.DMA,
        ],
    )
    def k(xw_hbm, xp_hbm, xs_hbm, Ww_hbm, Wp_hbm, Ws_hbm, out_hbm,
          idxw, idxp, idxs, sumb, bufp, bufs, sem):
        wid = lax.axis_index("s") * NC + lax.axis_index("c")
        base = wid * per_w

        def body(g, carry):
            off = base + g * CHUNK
            pltpu.sync_copy(xw_hbm.at[pl.ds(off, CHUNK)], idxw)
            pltpu.sync_copy(xp_hbm.at[pl.ds(off, CHUNK)], idxp)
            pltpu.sync_copy(xs_hbm.at[pl.ds(off, CHUNK)], idxs)
            cw = pltpu.async_copy(Ww_hbm.at[idxw], sumb, sem)
            cp = pltpu.async_copy(Wp_hbm.at[idxp], bufp, sem)
            cs = pltpu.async_copy(Ws_hbm.at[idxs], bufs, sem)
            cw.wait()
            cp.wait()
            cs.wait()

            def add_row(r, c2):
                for c4 in range(EMB // 16):
                    sl = pl.ds(c4 * 16, 16)
                    plsc.addupdate(sumb.at[r, sl], bufp[r, sl] + bufs[r, sl])
                return c2

            lax.fori_loop(0, CHUNK, add_row, 0)
            pltpu.sync_copy(sumb, out_hbm.at[pl.ds(off, CHUNK)])
            return carry

        lax.fori_loop(0, n_chunks, body, 0)

    return k(xw_f, xp_f, xs_f, Ww, Wp, Ws)


def _mlp(flat, W1, b1, W2, b2):
    B, K = flat.shape
    H = W1.shape[1]
    T = W2.shape[1]
    BM = 1024
    assert B % BM == 0

    def body(flat_ref, w1_ref, b1_ref, w2_ref, b2_ref, out_ref):
        h = jnp.tanh(
            jnp.dot(flat_ref[...], w1_ref[...], preferred_element_type=jnp.float32)
            + b1_ref[...]
        )
        out_ref[...] = (
            jnp.dot(h, w2_ref[...], preferred_element_type=jnp.float32) + b2_ref[...]
        )

    return pl.pallas_call(
        body,
        grid=(B // BM,),
        in_specs=[
            pl.BlockSpec((BM, K), lambda i: (i, 0)),
            pl.BlockSpec((K, H), lambda i: (0, 0)),
            pl.BlockSpec((1, H), lambda i: (0, 0)),
            pl.BlockSpec((H, T), lambda i: (0, 0)),
            pl.BlockSpec((1, T), lambda i: (0, 0)),
        ],
        out_specs=pl.BlockSpec((BM, T), lambda i: (i, 0)),
        out_shape=jax.ShapeDtypeStruct((B, T), jnp.float32),
    )(flat, W1, b1.reshape(1, H), W2, b2.reshape(1, T))


def kernel(xw, xp, xs, Ww, Wp, Ws, W1, b1, W2, b2):
    B, WIN = xw.shape
    e = _sc_gather_sum(
        xw.reshape(-1), xp.reshape(-1), xs.reshape(-1), Ww, Wp, Ws
    )
    flat = e.reshape(B, WIN * EMB)
    return _mlp(flat, W1, b1, W2, b2)
